# baseline (speedup 1.0000x reference)
import functools

import jax
import jax.numpy as jnp
from jax import lax
from jax.experimental import pallas as pl
from jax.experimental.pallas import tpu as pltpu
from jax.experimental.pallas import tpu_sc as plsc

NC = 2
NS = 16
NW = NC * NS
L = 16

IDXROW = 128
CH = 512
GPC = CH // L
SUB = CH // IDXROW
EPS = 1e-5


def _rsqrt(x):
    i = plsc.bitcast(x, jnp.int32)
    i = jnp.int32(0x5F3759DF) - (i >> 1)
    y = plsc.bitcast(i, jnp.float32)
    half = jnp.float32(0.5) * x
    for _ in range(3):
        y = y * (jnp.float32(1.5) - half * y * y)
    return y


def _make_sc_kernel(N, V, D):
    RW = N // NW
    J = RW // CH
    JROWS = RW // IDXROW
    mesh = plsc.VectorSubcoreMesh(core_axis_name="c", subcore_axis_name="s")

    @functools.partial(
        pl.kernel,
        out_type=jax.ShapeDtypeStruct((N, D), jnp.float32),
        mesh=mesh,
        compiler_params=pltpu.CompilerParams(
            use_tc_tiling_on_sc=False, needs_layout_passes=False
        ),
        scratch_types=[
            pltpu.VMEM((JROWS, IDXROW), jnp.int32),
            pltpu.VMEM((CH, D), jnp.float32),
            pltpu.VMEM((CH, D), jnp.float32),
            pltpu.VMEM((CH, D), jnp.float32),
            pltpu.VMEM((CH, D), jnp.float32),
            pltpu.VMEM((D, L), jnp.float32),
            pltpu.VMEM((D, L), jnp.float32),
            pltpu.SemaphoreType.DMA,
            pltpu.SemaphoreType.DMA,
            pltpu.SemaphoreType.DMA,
            pltpu.SemaphoreType.DMA,
        ],
    )
    def sc_embed_ln(idx_hbm, table_hbm, gb_hbm, bb_hbm, out_hbm,
                    idx_v, in0, in1, out0, out1, gbv, bbv,
                    is0, is1, os0, os1):
        wid = lax.axis_index("c") * NS + lax.axis_index("s")
        pltpu.sync_copy(idx_hbm.at[wid], idx_v)
        pltpu.sync_copy(gb_hbm, gbv)
        pltpu.sync_copy(bb_hbm, bbv)

        ins = (in0, in1)
        outs = (out0, out1)
        isems = (is0, is1)
        osems = (os0, os1)

        def issue_gather(j, b):
            for k in range(SUB):
                pltpu.async_copy(
                    table_hbm.at[idx_v.at[j * SUB + k]],
                    ins[b].at[pl.ds(k * IDXROW, IDXROW)],
                    isems[b],
                )

        def wait_gather(b):
            for k in range(SUB):
                pltpu.make_async_copy(
                    table_hbm.at[pl.ds(0, IDXROW)],
                    ins[b].at[pl.ds(k * IDXROW, IDXROW)],
                    isems[b],
                ).wait()

        def issue_out(j, b):
            pltpu.async_copy(
                outs[b],
                out_hbm.at[pl.ds(wid * RW + j * CH, CH)],
                osems[b],
            )

        def wait_out(b):
            pltpu.make_async_copy(
                outs[b],
                out_hbm.at[pl.ds(0, CH)],
                osems[b],
            ).wait()

        inv_d = jnp.float32(1.0 / D)

        def compute(b):
            in_ref = ins[b]
            out_ref = outs[b]

            @pl.loop(0, GPC)
            def _group(g):
                r = lax.iota(jnp.int32, L) + g * L
                v = []
                s = None
                q = None
                for d in range(D):
                    col = jnp.full((L,), d, dtype=jnp.int32)
                    x = plsc.load_gather(in_ref, [r, col])
                    v.append(x)
                    s = x if s is None else s + x
                    q = x * x if q is None else q + x * x
                mean = s * inv_d
                var = q * inv_d - mean * mean
                rstd = _rsqrt(var + jnp.float32(EPS))
                for d in range(D):
                    col = jnp.full((L,), d, dtype=jnp.int32)
                    y = (v[d] - mean) * rstd * gbv[d] + bbv[d]
                    plsc.store_scatter(out_ref, [r, col], y)

        issue_gather(0, 0)
        issue_gather(1, 1)
        for b in range(2):
            wait_gather(b)
            compute(b)
            issue_out(b, b)
            issue_gather(b + 2, b)

        @pl.loop(2, J - 2, step=2)
        def _main(j0):
            for b in range(2):
                j = j0 + b
                wait_gather(b)
                wait_out(b)
                compute(b)
                issue_out(j, b)
                issue_gather(j + 2, b)

        for b in range(2):
            wait_gather(b)
            wait_out(b)
            compute(b)
            issue_out(J - 2 + b, b)
        for b in range(2):
            wait_out(b)

    return sc_embed_ln


def kernel(input_ids, table, gamma, beta):
    B, H = input_ids.shape
    V, D = table.shape
    N = B * H
    idx = input_ids.astype(jnp.int32).reshape(NW, (N // NW) // IDXROW, IDXROW)
    gb = jnp.broadcast_to(gamma.astype(jnp.float32)[:, None], (D, L))
    bb = jnp.broadcast_to(beta.astype(jnp.float32)[:, None], (D, L))
    out = _make_sc_kernel(N, V, D)(idx, table, gb, bb)
    return out.reshape(B, H, D)


# speedup vs baseline: 1.3281x; 1.3281x over previous
import functools

import jax
import jax.numpy as jnp
from jax import lax
from jax.experimental import pallas as pl
from jax.experimental.pallas import tpu as pltpu
from jax.experimental.pallas import tpu_sc as plsc

NC = 2
NS = 16
NW = NC * NS
L = 16

IDXROW = 128
CH = 512
GPC = CH // L
SUB = CH // IDXROW
EPS = 1e-5


def _rsqrt(x):
    i = plsc.bitcast(x, jnp.int32)
    i = jnp.int32(0x5F3759DF) - (i >> 1)
    y = plsc.bitcast(i, jnp.float32)
    half = jnp.float32(0.5) * x
    for _ in range(3):
        y = y * (jnp.float32(1.5) - half * y * y)
    return y


def _make_sc_kernel(N, V, D):
    RW = N // NW
    J = RW // CH
    mesh = plsc.VectorSubcoreMesh(core_axis_name="c", subcore_axis_name="s")

    @functools.partial(
        pl.kernel,
        out_type=jax.ShapeDtypeStruct((N * D,), jnp.float32),
        mesh=mesh,
        compiler_params=pltpu.CompilerParams(
            use_tc_tiling_on_sc=False, needs_layout_passes=False
        ),
        scratch_types=[
            pltpu.VMEM((RW,), jnp.int32),
            pltpu.VMEM((CH, D), jnp.float32),
            pltpu.VMEM((CH, D), jnp.float32),
            pltpu.VMEM((CH * D,), jnp.float32),
            pltpu.VMEM((CH * D,), jnp.float32),
            pltpu.VMEM((D, L), jnp.float32),
            pltpu.VMEM((D, L), jnp.float32),
            pltpu.SemaphoreType.DMA,
            pltpu.SemaphoreType.DMA,
            pltpu.SemaphoreType.DMA,
            pltpu.SemaphoreType.DMA,
        ],
    )
    def sc_embed_ln(idx_hbm, table_hbm, gb_hbm, bb_hbm, out_hbm,
                    idx_v, in0, in1, out0, out1, gbv, bbv,
                    is0, is1, os0, os1):
        wid = lax.axis_index("c") * NS + lax.axis_index("s")
        pltpu.sync_copy(idx_hbm.at[pl.ds(wid * RW, RW)], idx_v)
        pltpu.sync_copy(gb_hbm, gbv)
        pltpu.sync_copy(bb_hbm, bbv)

        ins = (in0, in1)
        outs = (out0, out1)
        isems = (is0, is1)
        osems = (os0, os1)

        def issue_gather(j, b):
            for k in range(SUB):
                pltpu.async_copy(
                    table_hbm.at[idx_v.at[pl.ds((j * SUB + k) * IDXROW, IDXROW)]],
                    ins[b].at[pl.ds(k * IDXROW, IDXROW)],
                    isems[b],
                )

        def wait_gather(b):
            for k in range(SUB):
                pltpu.make_async_copy(
                    table_hbm.at[pl.ds(0, IDXROW)],
                    ins[b].at[pl.ds(k * IDXROW, IDXROW)],
                    isems[b],
                ).wait()

        def issue_out(j, b):
            pltpu.async_copy(
                outs[b],
                out_hbm.at[pl.ds((wid * RW + j * CH) * D, CH * D)],
                osems[b],
            )

        def wait_out(b):
            pltpu.make_async_copy(
                outs[b],
                out_hbm.at[pl.ds(0, CH * D)],
                osems[b],
            ).wait()

        inv_d = jnp.float32(1.0 / D)

        def compute(b):
            in_ref = ins[b]
            out_ref = outs[b]

            @pl.loop(0, GPC)
            def _group(g):
                r = lax.iota(jnp.int32, L) + g * L
                flat = r * D
                v = []
                s = None
                q = None
                for d in range(D):
                    col = jnp.full((L,), d, dtype=jnp.int32)
                    x = plsc.load_gather(in_ref, [r, col])
                    v.append(x)
                    s = x if s is None else s + x
                    q = x * x if q is None else q + x * x
                mean = s * inv_d
                var = q * inv_d - mean * mean
                rstd = _rsqrt(var + jnp.float32(EPS))
                for d in range(D):
                    y = (v[d] - mean) * rstd * gbv[d] + bbv[d]
                    plsc.store_scatter(out_ref, [flat + d], y)

        issue_gather(0, 0)
        issue_gather(1, 1)
        for b in range(2):
            wait_gather(b)
            compute(b)
            issue_out(b, b)
            issue_gather(b + 2, b)

        @pl.loop(2, J - 2, step=2)
        def _main(j0):
            for b in range(2):
                j = j0 + b
                wait_gather(b)
                wait_out(b)
                compute(b)
                issue_out(j, b)
                issue_gather(j + 2, b)

        for b in range(2):
            wait_gather(b)
            wait_out(b)
            compute(b)
            issue_out(J - 2 + b, b)
        for b in range(2):
            wait_out(b)

    return sc_embed_ln


def kernel(input_ids, table, gamma, beta):
    B, H = input_ids.shape
    V, D = table.shape
    N = B * H
    idx = input_ids.astype(jnp.int32).reshape(N)
    gb = jnp.broadcast_to(gamma.astype(jnp.float32)[:, None], (D, L))
    bb = jnp.broadcast_to(beta.astype(jnp.float32)[:, None], (D, L))
    out = _make_sc_kernel(N, V, D)(idx, table, gb, bb)
    return out.reshape(B, H, D)
